# baseline (speedup 1.0000x reference)
import jax
import jax.numpy as jnp
from jax.experimental import pallas as pl


def _dc_block(pred_ref, meas_ref, mask_ref, out_ref):
    out_ref[...] = jnp.where(mask_ref[...], meas_ref[...], pred_ref[...])


def kernel(k_pred, k_meas, mask):
    B, H, W = k_pred.shape
    blk = 4
    specs = [pl.BlockSpec((blk, H, W), lambda i: (i, 0, 0)) for _ in range(3)]
    return pl.pallas_call(
        _dc_block,
        grid=(B // blk,),
        in_specs=specs,
        out_specs=pl.BlockSpec((blk, H, W), lambda i: (i, 0, 0)),
        out_shape=jax.ShapeDtypeStruct((B, H, W), jnp.float32),
    )(k_pred, k_meas, mask)


# speedup vs baseline: 1.3575x; 1.3575x over previous
import jax
import jax.numpy as jnp
from jax.experimental import pallas as pl


def _dc_block(pred_ref, meas_ref, mask_ref, out_ref):
    out_ref[...] = jnp.where(mask_ref[...] != 0, meas_ref[...], pred_ref[...])


def kernel(k_pred, k_meas, mask):
    B, H, W = k_pred.shape
    mask8 = mask.view(jnp.int8)
    blk = 4
    specs = [pl.BlockSpec((blk, H, W), lambda i: (i, 0, 0)) for _ in range(3)]
    return pl.pallas_call(
        _dc_block,
        grid=(B // blk,),
        in_specs=specs,
        out_specs=pl.BlockSpec((blk, H, W), lambda i: (i, 0, 0)),
        out_shape=jax.ShapeDtypeStruct((B, H, W), jnp.float32),
    )(k_pred, k_meas, mask8)
